# baseline (speedup 1.0000x reference)
import jax
import jax.numpy as jnp
from jax.experimental import pallas as pl


def _copy_kernel(x_ref, o_ref):
    o_ref[...] = x_ref[...]


def kernel(mat0, mat1, idx_from0, idx_to0, idx_from1, idx_to1):
    L, _, C = mat0.shape
    num_rows = idx_to0.shape[0] + idx_to1.shape[0]
    out = jnp.zeros((L, num_rows, C), dtype=mat0.dtype)
    out = out.at[:, idx_to0].set(jnp.take(mat0, idx_from0, axis=1))
    out = out.at[:, idx_to1].set(jnp.take(mat1, idx_from1, axis=1))
    first = pl.pallas_call(
        _copy_kernel,
        out_shape=jax.ShapeDtypeStruct((8, 128), out.dtype),
    )(out[0, :16, :64].reshape(8, 128))
    out = out.at[0, :16, :64].set(first.reshape(16, 64))
    return out


# speedup vs baseline: 18.8924x; 18.8924x over previous
import functools

import jax
import jax.numpy as jnp
from jax import lax
from jax.experimental import pallas as pl
from jax.experimental.pallas import tpu as pltpu
from jax.experimental.pallas import tpu_sc as plsc

L, M, C = 2, 200000, 64
NI = 100000
R = 2 * NI
NW = 32
OWN = R // NW
ICH = 4000
NICH = NI // ICH
VPC = ICH // 16
LISTN = 6528
RCH = 128
GRP = 6
ROWBUF = GRP * RCH
MAXCH = (OWN + RCH - 1) // RCH


def _sc_body(a0, a1, if0, it0, if1, it1, zhbm, out, win, tbuf, fbuf,
             s0, d0, s1, d1, dz, rowbuf, zbuf, semg, sems):
    wid = lax.axis_index("s") * 2 + lax.axis_index("c")
    base = wid * OWN
    lanes = lax.iota(jnp.int32, 16)
    neg1 = jnp.zeros((16,), jnp.int32) - 1

    def init_body(k, carry):
        win[pl.ds(k * 16, 16)] = neg1
        return carry

    lax.fori_loop(0, (OWN + 15) // 16, init_body, 0)

    for s, it_hbm, if_hbm in ((0, it0, if0), (1, it1, if1)):

        def chunk_body(ch, carry):
            pltpu.sync_copy(it_hbm.at[pl.ds(ch * ICH, ICH)], tbuf)
            pltpu.sync_copy(if_hbm.at[pl.ds(ch * ICH, ICH)], fbuf)

            def vreg_body(v, c2):
                t = tbuf[pl.ds(v * 16, 16)]
                f = fbuf[pl.ds(v * 16, 16)]
                rel = t - base
                inb = (rel >= 0) & (rel < OWN)
                g = f + s * M
                _, lastm = plsc.scan_count(rel, mask=inb)
                plsc.store_scatter(win, [rel], g, mask=lastm & inb)
                return c2

            lax.fori_loop(0, VPC, vreg_body, 0)
            return carry

        lax.fori_loop(0, NICH, chunk_body, 0)

    pltpu.sync_copy(zhbm, zbuf)

    for l in range(L):

        def comp_body(v, carry):
            c0, c1, cz = carry
            g = win[pl.ds(v * 16, 16)]
            pos = v * 16 + lanes
            valid = pos < OWN
            dest = base + pos + l * R
            isz = g < 0
            is1 = g >= M
            m0 = valid & (~isz) & (~is1)
            m1 = valid & is1
            mz = valid & isz
            plsc.store_compressed(s0.at[pl.ds(c0, 16)], g + l * M, mask=m0)
            plsc.store_compressed(d0.at[pl.ds(c0, 16)], dest, mask=m0)
            plsc.store_compressed(s1.at[pl.ds(c1, 16)], g + (l - 1) * M, mask=m1)
            plsc.store_compressed(d1.at[pl.ds(c1, 16)], dest, mask=m1)
            plsc.store_compressed(dz.at[pl.ds(cz, 16)], dest, mask=mz)
            c0 = c0 + jnp.max(plsc.all_reduce_population_count(m0))
            c1 = c1 + jnp.max(plsc.all_reduce_population_count(m1))
            cz = cz + jnp.max(plsc.all_reduce_population_count(mz))
            return c0, c1, cz

        nv = (OWN + 15) // 16
        c0, c1, cz = lax.fori_loop(0, nv, comp_body, (jnp.int32(0), jnp.int32(0), jnp.int32(0)))

        for svec, dvec, cnt in ((s0, d0, c0), (s1, d1, c1), (None, dz, cz)):
            dpad = jnp.zeros((16,), jnp.int32) + dvec[pl.ds(0, 16)][0]
            spad = (
                None
                if svec is None
                else jnp.zeros((16,), jnp.int32) + svec[pl.ds(0, 16)][0]
            )
            for k in range(RCH // 16):
                dvec[pl.ds(cnt + k * 16, 16)] = dpad
                if svec is not None:
                    svec[pl.ds(cnt + k * 16, 16)] = spad

        for src_hbm, svec, dvec, cnt in (
            (a0, s0, d0, c0), (a1, s1, d1, c1), (None, None, dz, cz)
        ):
            nch = (cnt + RCH - 1) // RCH
            ngrp = (nch + GRP - 1) // GRP

            def grp_body(gi, carry):
                if src_hbm is not None:
                    for k in range(GRP):
                        cidx = gi * GRP + k

                        @pl.when(cidx < nch)
                        def _():
                            pltpu.make_async_copy(
                                src_hbm.at[svec.at[pl.ds(cidx * RCH, RCH)]],
                                rowbuf.at[pl.ds(k * RCH, RCH)],
                                semg,
                            ).start()

                    for k in range(GRP):
                        cidx = gi * GRP + k

                        @pl.when(cidx < nch)
                        def _():
                            pltpu.make_async_copy(
                                src_hbm.at[svec.at[pl.ds(cidx * RCH, RCH)]],
                                rowbuf.at[pl.ds(k * RCH, RCH)],
                                semg,
                            ).wait()

                for k in range(GRP):
                    cidx = gi * GRP + k
                    sbuf = zbuf if src_hbm is None else rowbuf.at[pl.ds(k * RCH, RCH)]

                    @pl.when(cidx < nch)
                    def _():
                        pltpu.make_async_copy(
                            sbuf,
                            out.at[dvec.at[pl.ds(cidx * RCH, RCH)]],
                            sems,
                        ).start()

                for k in range(GRP):
                    cidx = gi * GRP + k
                    sbuf = zbuf if src_hbm is None else rowbuf.at[pl.ds(k * RCH, RCH)]

                    @pl.when(cidx < nch)
                    def _():
                        pltpu.make_async_copy(
                            sbuf,
                            out.at[dvec.at[pl.ds(cidx * RCH, RCH)]],
                            sems,
                        ).wait()

                return carry

            lax.fori_loop(0, ngrp, grp_body, 0)


def _build():
    mesh = plsc.VectorSubcoreMesh(core_axis_name="c", subcore_axis_name="s")
    return pl.kernel(
        _sc_body,
        out_type=jax.ShapeDtypeStruct((L * R, C), jnp.float32),
        mesh=mesh,
        scratch_types=[
            pltpu.VMEM((((OWN + 15) // 16) * 16,), jnp.int32),
            pltpu.VMEM((ICH,), jnp.int32),
            pltpu.VMEM((ICH,), jnp.int32),
            pltpu.VMEM((LISTN,), jnp.int32),
            pltpu.VMEM((LISTN,), jnp.int32),
            pltpu.VMEM((LISTN,), jnp.int32),
            pltpu.VMEM((LISTN,), jnp.int32),
            pltpu.VMEM((LISTN,), jnp.int32),
            pltpu.VMEM((ROWBUF, C), jnp.float32),
            pltpu.VMEM((RCH, C), jnp.float32),
            pltpu.SemaphoreType.DMA,
            pltpu.SemaphoreType.DMA,
        ],
        compiler_params=pltpu.CompilerParams(
            needs_layout_passes=False, use_tc_tiling_on_sc=False
        ),
    )


def kernel(mat0, mat1, idx_from0, idx_to0, idx_from1, idx_to1):
    a0 = mat0.reshape(L * M, C)
    a1 = mat1.reshape(L * M, C)
    z = jnp.zeros((RCH, C), jnp.float32)
    sck = _build()
    outflat = sck(
        a0,
        a1,
        idx_from0.astype(jnp.int32),
        idx_to0.astype(jnp.int32),
        idx_from1.astype(jnp.int32),
        idx_to1.astype(jnp.int32),
        z,
    )
    return outflat.reshape(L, R, C)


# speedup vs baseline: 18.9273x; 1.0018x over previous
import jax
import jax.numpy as jnp
from jax import lax
from jax.experimental import pallas as pl
from jax.experimental.pallas import tpu as pltpu
from jax.experimental.pallas import tpu_sc as plsc

L, M, C = 2, 200000, 64
NI = 100000
R = 2 * NI
NW = 32
OWN = R // NW
ICH = 4000
NICH = NI // ICH
VPC = ICH // 16
LISTN = 6528
RCH = 128
GRP = 6
ROWBUF = GRP * RCH


def _sc_body(a0, a1, if0, it0, if1, it1, zhbm, out, win, tbuf, fbuf,
             s0, d0, s1, d1, dz, rowbuf, zbuf, semg, sems):
    wid = lax.axis_index("s") * 2 + lax.axis_index("c")
    base = wid * OWN
    lanes = lax.iota(jnp.int32, 16)
    neg1 = jnp.zeros((16,), jnp.int32) - 1

    def init_body(k, carry):
        win[pl.ds(k * 16, 16)] = neg1
        return carry

    lax.fori_loop(0, (OWN + 15) // 16, init_body, 0)

    for s, it_hbm, if_hbm in ((0, it0, if0), (1, it1, if1)):

        def chunk_body(ch, carry):
            pltpu.sync_copy(it_hbm.at[pl.ds(ch * ICH, ICH)], tbuf)
            pltpu.sync_copy(if_hbm.at[pl.ds(ch * ICH, ICH)], fbuf)

            def vreg_body(v, c2):
                t = tbuf[pl.ds(v * 16, 16)]
                f = fbuf[pl.ds(v * 16, 16)]
                rel = t - base
                inb = (rel >= 0) & (rel < OWN)
                g = f + s * M
                _, lastm = plsc.scan_count(rel, mask=inb)
                plsc.store_scatter(win, [rel], g, mask=lastm & inb)
                return c2

            lax.fori_loop(0, VPC, vreg_body, 0)
            return carry

        lax.fori_loop(0, NICH, chunk_body, 0)

    pltpu.sync_copy(zhbm, zbuf)

    def comp_body(v, carry):
        c0, c1, cz = carry
        g = win[pl.ds(v * 16, 16)]
        pos = v * 16 + lanes
        valid = pos < OWN
        dest = base + pos
        isz = g < 0
        is1 = g >= M
        m0 = valid & (~isz) & (~is1)
        m1 = valid & is1
        mz = valid & isz
        plsc.store_compressed(s0.at[pl.ds(c0, 16)], g, mask=m0)
        plsc.store_compressed(d0.at[pl.ds(c0, 16)], dest, mask=m0)
        plsc.store_compressed(s1.at[pl.ds(c1, 16)], g - M, mask=m1)
        plsc.store_compressed(d1.at[pl.ds(c1, 16)], dest, mask=m1)
        plsc.store_compressed(dz.at[pl.ds(cz, 16)], dest, mask=mz)
        c0 = c0 + jnp.max(plsc.all_reduce_population_count(m0))
        c1 = c1 + jnp.max(plsc.all_reduce_population_count(m1))
        cz = cz + jnp.max(plsc.all_reduce_population_count(mz))
        return c0, c1, cz

    nv = (OWN + 15) // 16
    c0, c1, cz = lax.fori_loop(
        0, nv, comp_body, (jnp.int32(0), jnp.int32(0), jnp.int32(0))
    )

    for svec, dvec, cnt in ((s0, d0, c0), (s1, d1, c1), (None, dz, cz)):
        dpad = jnp.zeros((16,), jnp.int32) + dvec[pl.ds(0, 16)][0]
        spad = (
            None
            if svec is None
            else jnp.zeros((16,), jnp.int32) + svec[pl.ds(0, 16)][0]
        )
        for k in range(RCH // 16):
            dvec[pl.ds(cnt + k * 16, 16)] = dpad
            if svec is not None:
                svec[pl.ds(cnt + k * 16, 16)] = spad

    for l in range(L):
        for src_hbm, svec, dvec, cnt in (
            (a0, s0, d0, c0), (a1, s1, d1, c1), (None, None, dz, cz)
        ):
            nch = (cnt + RCH - 1) // RCH
            ngrp = (nch + GRP - 1) // GRP

            def grp_body(gi, carry):
                if src_hbm is not None:
                    for k in range(GRP):
                        cidx = gi * GRP + k

                        @pl.when(cidx < nch)
                        def _():
                            pltpu.make_async_copy(
                                src_hbm.at[l].at[svec.at[pl.ds(cidx * RCH, RCH)]],
                                rowbuf.at[pl.ds(k * RCH, RCH)],
                                semg,
                            ).start()

                    for k in range(GRP):
                        cidx = gi * GRP + k

                        @pl.when(cidx < nch)
                        def _():
                            pltpu.make_async_copy(
                                src_hbm.at[l].at[svec.at[pl.ds(cidx * RCH, RCH)]],
                                rowbuf.at[pl.ds(k * RCH, RCH)],
                                semg,
                            ).wait()

                for k in range(GRP):
                    cidx = gi * GRP + k
                    sbuf = zbuf if src_hbm is None else rowbuf.at[pl.ds(k * RCH, RCH)]

                    @pl.when(cidx < nch)
                    def _():
                        pltpu.make_async_copy(
                            sbuf,
                            out.at[l].at[dvec.at[pl.ds(cidx * RCH, RCH)]],
                            sems,
                        ).start()

                for k in range(GRP):
                    cidx = gi * GRP + k
                    sbuf = zbuf if src_hbm is None else rowbuf.at[pl.ds(k * RCH, RCH)]

                    @pl.when(cidx < nch)
                    def _():
                        pltpu.make_async_copy(
                            sbuf,
                            out.at[l].at[dvec.at[pl.ds(cidx * RCH, RCH)]],
                            sems,
                        ).wait()

                return carry

            lax.fori_loop(0, ngrp, grp_body, 0)


def _build():
    mesh = plsc.VectorSubcoreMesh(core_axis_name="c", subcore_axis_name="s")
    return pl.kernel(
        _sc_body,
        out_type=jax.ShapeDtypeStruct((L, R, C), jnp.float32),
        mesh=mesh,
        scratch_types=[
            pltpu.VMEM((((OWN + 15) // 16) * 16,), jnp.int32),
            pltpu.VMEM((ICH,), jnp.int32),
            pltpu.VMEM((ICH,), jnp.int32),
            pltpu.VMEM((LISTN,), jnp.int32),
            pltpu.VMEM((LISTN,), jnp.int32),
            pltpu.VMEM((LISTN,), jnp.int32),
            pltpu.VMEM((LISTN,), jnp.int32),
            pltpu.VMEM((LISTN,), jnp.int32),
            pltpu.VMEM((ROWBUF, C), jnp.float32),
            pltpu.VMEM((RCH, C), jnp.float32),
            pltpu.SemaphoreType.DMA,
            pltpu.SemaphoreType.DMA,
        ],
        compiler_params=pltpu.CompilerParams(
            needs_layout_passes=False, use_tc_tiling_on_sc=False
        ),
    )


def kernel(mat0, mat1, idx_from0, idx_to0, idx_from1, idx_to1):
    z = jnp.zeros((RCH, C), jnp.float32)
    sck = _build()
    return sck(
        mat0,
        mat1,
        idx_from0.astype(jnp.int32),
        idx_to0.astype(jnp.int32),
        idx_from1.astype(jnp.int32),
        idx_to1.astype(jnp.int32),
        z,
    )


# speedup vs baseline: 20.8220x; 1.1001x over previous
import jax
import jax.numpy as jnp
from jax import lax
from jax.experimental import pallas as pl
from jax.experimental.pallas import tpu as pltpu
from jax.experimental.pallas import tpu_sc as plsc

L, M, C = 2, 200000, 64
NI = 100000
R = 2 * NI
NW = 32
OWN = R // NW
ICH = 2000
NICH = NI // ICH
U = 5
GU = ICH // 16 // U
LISTN = 6528
RCH = 128
GRP = 4
ROWBUF = 2 * GRP * RCH


def _sc_body(a0, a1, if0, it0, if1, it1, zhbm, out, win, tbuf, fbuf,
             s0, d0, s1, d1, dz, rowbuf, zbuf, semt, semg, sems, semz):
    wid = lax.axis_index("s") * 2 + lax.axis_index("c")
    base = wid * OWN
    lanes = lax.iota(jnp.int32, 16)
    neg1 = jnp.zeros((16,), jnp.int32) - 1

    def init_body(k, carry):
        win[pl.ds(k * 16, 16)] = neg1
        return carry

    lax.fori_loop(0, (OWN + 15) // 16, init_body, 0)
    pltpu.sync_copy(zhbm, zbuf)

    for s, it_hbm, if_hbm in ((0, it0, if0), (1, it1, if1)):

        def stream(ch, off):
            return (
                pltpu.make_async_copy(
                    it_hbm.at[pl.ds(ch * ICH, ICH)], tbuf.at[pl.ds(off, ICH)], semt
                ),
                pltpu.make_async_copy(
                    if_hbm.at[pl.ds(ch * ICH, ICH)], fbuf.at[pl.ds(off, ICH)], semt
                ),
            )

        for cp in stream(0, 0):
            cp.start()

        def chunk_body(ch, carry):
            off = (ch & 1) * ICH
            for cp in stream(ch, off):
                cp.wait()

            @pl.when(ch + 1 < NICH)
            def _():
                for cp in stream(ch + 1, ((ch + 1) & 1) * ICH):
                    cp.start()

            def vreg_body(v, c2):
                for u in range(U):
                    o = off + (v * U + u) * 16
                    t = tbuf[pl.ds(o, 16)]
                    f = fbuf[pl.ds(o, 16)]
                    rel = t - base
                    inb = (rel >= 0) & (rel < OWN)
                    g = f + s * M
                    _, lastm = plsc.scan_count(rel, mask=inb)
                    plsc.store_scatter(win, [rel], g, mask=lastm & inb)
                return c2

            lax.fori_loop(0, GU, vreg_body, 0)
            return carry

        lax.fori_loop(0, NICH, chunk_body, 0)

    def comp_body(v, carry):
        c0, c1, cz = carry
        g = win[pl.ds(v * 16, 16)]
        pos = v * 16 + lanes
        valid = pos < OWN
        dest = base + pos
        isz = g < 0
        is1 = g >= M
        m0 = valid & (~isz) & (~is1)
        m1 = valid & is1
        mz = valid & isz
        plsc.store_compressed(s0.at[pl.ds(c0, 16)], g, mask=m0)
        plsc.store_compressed(d0.at[pl.ds(c0, 16)], dest, mask=m0)
        plsc.store_compressed(s1.at[pl.ds(c1, 16)], g - M, mask=m1)
        plsc.store_compressed(d1.at[pl.ds(c1, 16)], dest, mask=m1)
        plsc.store_compressed(dz.at[pl.ds(cz, 16)], dest, mask=mz)
        c0 = c0 + jnp.max(plsc.all_reduce_population_count(m0))
        c1 = c1 + jnp.max(plsc.all_reduce_population_count(m1))
        cz = cz + jnp.max(plsc.all_reduce_population_count(mz))
        return c0, c1, cz

    nv = (OWN + 15) // 16
    c0, c1, cz = lax.fori_loop(
        0, nv, comp_body, (jnp.int32(0), jnp.int32(0), jnp.int32(0))
    )

    for svec, dvec, cnt in ((s0, d0, c0), (s1, d1, c1), (None, dz, cz)):
        dpad = jnp.zeros((16,), jnp.int32) + dvec[pl.ds(0, 16)][0]
        spad = (
            None
            if svec is None
            else jnp.zeros((16,), jnp.int32) + svec[pl.ds(0, 16)][0]
        )
        for k in range(RCH // 16):
            dvec[pl.ds(cnt + k * 16, 16)] = dpad
            if svec is not None:
                svec[pl.ds(cnt + k * 16, 16)] = spad

    nchz = (cz + RCH - 1) // RCH

    def zcopy(l, ci):
        return pltpu.make_async_copy(
            zbuf, out.at[l].at[dz.at[pl.ds(ci * RCH, RCH)]], semz
        )

    def zfire_body(ci, carry):
        for l in range(L):
            zcopy(l, ci).start()
        return carry

    lax.fori_loop(0, nchz, zfire_body, 0)

    for l in range(L):
        for src_hbm, svec, dvec, cnt in ((a0, s0, d0, c0), (a1, s1, d1, c1)):
            nch = (cnt + RCH - 1) // RCH
            ngrp = (nch + GRP - 1) // GRP

            def gcopy(ci, k, p):
                return pltpu.make_async_copy(
                    src_hbm.at[l].at[svec.at[pl.ds(ci * RCH, RCH)]],
                    rowbuf.at[pl.ds((p * GRP + k) * RCH, RCH)],
                    semg,
                )

            def scopy(ci, k, p):
                return pltpu.make_async_copy(
                    rowbuf.at[pl.ds((p * GRP + k) * RCH, RCH)],
                    out.at[l].at[dvec.at[pl.ds(ci * RCH, RCH)]],
                    sems,
                )

            def grp_body(gi, carry):
                @pl.when(gi >= 2)
                def _():
                    for k in range(GRP):
                        ci = (gi - 2) * GRP + k

                        @pl.when(ci < nch)
                        def _():
                            scopy(ci, k, gi & 1).wait()

                @pl.when(gi < ngrp)
                def _():
                    for k in range(GRP):
                        ci = gi * GRP + k

                        @pl.when(ci < nch)
                        def _():
                            gcopy(ci, k, gi & 1).start()

                    for k in range(GRP):
                        ci = gi * GRP + k

                        @pl.when(ci < nch)
                        def _():
                            gcopy(ci, k, gi & 1).wait()

                    for k in range(GRP):
                        ci = gi * GRP + k

                        @pl.when(ci < nch)
                        def _():
                            scopy(ci, k, gi & 1).start()

                return carry

            lax.fori_loop(0, ngrp + 2, grp_body, 0)

    def zdrain_body(ci, carry):
        for l in range(L):
            zcopy(l, ci).wait()
        return carry

    lax.fori_loop(0, nchz, zdrain_body, 0)


def _build():
    mesh = plsc.VectorSubcoreMesh(core_axis_name="c", subcore_axis_name="s")
    return pl.kernel(
        _sc_body,
        out_type=jax.ShapeDtypeStruct((L, R, C), jnp.float32),
        mesh=mesh,
        scratch_types=[
            pltpu.VMEM((((OWN + 15) // 16) * 16,), jnp.int32),
            pltpu.VMEM((2 * ICH,), jnp.int32),
            pltpu.VMEM((2 * ICH,), jnp.int32),
            pltpu.VMEM((LISTN,), jnp.int32),
            pltpu.VMEM((LISTN,), jnp.int32),
            pltpu.VMEM((LISTN,), jnp.int32),
            pltpu.VMEM((LISTN,), jnp.int32),
            pltpu.VMEM((LISTN,), jnp.int32),
            pltpu.VMEM((ROWBUF, C), jnp.float32),
            pltpu.VMEM((RCH, C), jnp.float32),
            pltpu.SemaphoreType.DMA,
            pltpu.SemaphoreType.DMA,
            pltpu.SemaphoreType.DMA,
            pltpu.SemaphoreType.DMA,
        ],
        compiler_params=pltpu.CompilerParams(
            needs_layout_passes=False, use_tc_tiling_on_sc=False
        ),
    )


def kernel(mat0, mat1, idx_from0, idx_to0, idx_from1, idx_to1):
    z = jnp.zeros((RCH, C), jnp.float32)
    sck = _build()
    return sck(
        mat0,
        mat1,
        idx_from0.astype(jnp.int32),
        idx_to0.astype(jnp.int32),
        idx_from1.astype(jnp.int32),
        idx_to1.astype(jnp.int32),
        z,
    )


# speedup vs baseline: 24.9551x; 1.1985x over previous
import jax
import jax.numpy as jnp
from jax import lax
from jax.experimental import pallas as pl
from jax.experimental.pallas import tpu as pltpu
from jax.experimental.pallas import tpu_sc as plsc

L, M, C = 2, 200000, 64
NI = 100000
R = 2 * NI
NW = 32
OWN = R // NW
ICH = 2000
NICH = NI // ICH
U = 5
GU = ICH // 16 // U
LISTN = 6528
RCH = 128
GRP = 4
ROWBUF = 2 * GRP * RCH
NL = 5


def _wid():
    return lax.axis_index("s") * 2 + lax.axis_index("c")


def _lists_body(if0, it0, if1, it1, lst, cnto, win, tbuf, fbuf,
                s0, d0, s1, d1, dz, cbuf, semt):
    wid = _wid()
    base = wid * OWN
    lanes = lax.iota(jnp.int32, 16)
    neg1 = jnp.zeros((16,), jnp.int32) - 1

    def init_body(k, carry):
        win[pl.ds(k * 16, 16)] = neg1
        return carry

    lax.fori_loop(0, (OWN + 15) // 16, init_body, 0)

    for s, it_hbm, if_hbm in ((0, it0, if0), (1, it1, if1)):

        def stream(ch, off):
            return (
                pltpu.make_async_copy(
                    it_hbm.at[pl.ds(ch * ICH, ICH)], tbuf.at[pl.ds(off, ICH)], semt
                ),
                pltpu.make_async_copy(
                    if_hbm.at[pl.ds(ch * ICH, ICH)], fbuf.at[pl.ds(off, ICH)], semt
                ),
            )

        for cp in stream(0, 0):
            cp.start()

        def chunk_body(ch, carry):
            off = (ch & 1) * ICH
            for cp in stream(ch, off):
                cp.wait()

            @pl.when(ch + 1 < NICH)
            def _():
                for cp in stream(ch + 1, ((ch + 1) & 1) * ICH):
                    cp.start()

            def vreg_body(v, c2):
                for u in range(U):
                    o = off + (v * U + u) * 16
                    t = tbuf[pl.ds(o, 16)]
                    f = fbuf[pl.ds(o, 16)]
                    rel = t - base
                    inb = (rel >= 0) & (rel < OWN)
                    g = f + s * M
                    _, lastm = plsc.scan_count(rel, mask=inb)
                    plsc.store_scatter(win, [rel], g, mask=lastm & inb)
                return c2

            lax.fori_loop(0, GU, vreg_body, 0)
            return carry

        lax.fori_loop(0, NICH, chunk_body, 0)

    def comp_body(v, carry):
        c0, c1, cz = carry
        g = win[pl.ds(v * 16, 16)]
        pos = v * 16 + lanes
        valid = pos < OWN
        dest = base + pos
        isz = g < 0
        is1 = g >= M
        m0 = valid & (~isz) & (~is1)
        m1 = valid & is1
        mz = valid & isz
        plsc.store_compressed(s0.at[pl.ds(c0, 16)], g, mask=m0)
        plsc.store_compressed(d0.at[pl.ds(c0, 16)], dest, mask=m0)
        plsc.store_compressed(s1.at[pl.ds(c1, 16)], g - M, mask=m1)
        plsc.store_compressed(d1.at[pl.ds(c1, 16)], dest, mask=m1)
        plsc.store_compressed(dz.at[pl.ds(cz, 16)], dest, mask=mz)
        c0 = c0 + jnp.max(plsc.all_reduce_population_count(m0))
        c1 = c1 + jnp.max(plsc.all_reduce_population_count(m1))
        cz = cz + jnp.max(plsc.all_reduce_population_count(mz))
        return c0, c1, cz

    nv = (OWN + 15) // 16
    c0, c1, cz = lax.fori_loop(
        0, nv, comp_body, (jnp.int32(0), jnp.int32(0), jnp.int32(0))
    )

    for svec, dvec, cnt in ((s0, d0, c0), (s1, d1, c1), (None, dz, cz)):
        dpad = jnp.zeros((16,), jnp.int32) + dvec[pl.ds(0, 16)][0]
        spad = (
            None
            if svec is None
            else jnp.zeros((16,), jnp.int32) + svec[pl.ds(0, 16)][0]
        )
        for k in range(RCH // 16):
            dvec[pl.ds(cnt + k * 16, 16)] = dpad
            if svec is not None:
                svec[pl.ds(cnt + k * 16, 16)] = spad

    for i, buf in enumerate((s0, d0, s1, d1, dz)):
        pltpu.sync_copy(buf, lst.at[wid].at[i])
    cv = jnp.where(lanes == 0, c0, jnp.where(lanes == 1, c1, jnp.where(lanes == 2, cz, 0)))
    cbuf[...] = cv
    pltpu.sync_copy(cbuf.at[pl.ds(0, 8)], cnto.at[wid])


def _move_body(a0, a1, lst, cnt, zhbm, out, s0, d0, s1, d1, dz, cbuf,
               rowbuf, zbuf, semg, sems, semz):
    wid = _wid()

    pltpu.sync_copy(zhbm, zbuf)
    for i, buf in enumerate((s0, d0, s1, d1, dz)):
        pltpu.sync_copy(lst.at[wid].at[i], buf)
    pltpu.sync_copy(cnt.at[wid], cbuf.at[pl.ds(0, 8)])
    cv = cbuf[pl.ds(0, 16)]
    c0 = cv[0]
    c1 = cv[1]
    cz = cv[2]

    nchz = (cz + RCH - 1) >> 7

    def zcopy(l, ci):
        return pltpu.make_async_copy(
            zbuf, out.at[l].at[dz.at[pl.ds(ci * RCH, RCH)]], semz
        )

    def zfire_body(ci, carry):
        for l in range(L):
            zcopy(l, ci).start()
        return carry

    lax.fori_loop(0, nchz, zfire_body, 0)

    for l in range(L):
        for src_hbm, svec, dvec, cn in ((a0, s0, d0, c0), (a1, s1, d1, c1)):
            nch = (cn + RCH - 1) >> 7
            ngrp = (nch + GRP - 1) // GRP

            def gcopy(ci, k, p):
                return pltpu.make_async_copy(
                    src_hbm.at[l].at[svec.at[pl.ds(ci * RCH, RCH)]],
                    rowbuf.at[pl.ds((p * GRP + k) * RCH, RCH)],
                    semg,
                )

            def scopy(ci, k, p):
                return pltpu.make_async_copy(
                    rowbuf.at[pl.ds((p * GRP + k) * RCH, RCH)],
                    out.at[l].at[dvec.at[pl.ds(ci * RCH, RCH)]],
                    sems,
                )

            def grp_body(gi, carry):
                @pl.when(gi >= 2)
                def _():
                    for k in range(GRP):
                        ci = (gi - 2) * GRP + k

                        @pl.when(ci < nch)
                        def _():
                            scopy(ci, k, gi & 1).wait()

                @pl.when(gi < ngrp)
                def _():
                    for k in range(GRP):
                        ci = gi * GRP + k

                        @pl.when(ci < nch)
                        def _():
                            gcopy(ci, k, gi & 1).start()

                    for k in range(GRP):
                        ci = gi * GRP + k

                        @pl.when(ci < nch)
                        def _():
                            gcopy(ci, k, gi & 1).wait()

                    for k in range(GRP):
                        ci = gi * GRP + k

                        @pl.when(ci < nch)
                        def _():
                            scopy(ci, k, gi & 1).start()

                return carry

            lax.fori_loop(0, ngrp + 2, grp_body, 0)

    def zdrain_body(ci, carry):
        for l in range(L):
            zcopy(l, ci).wait()
        return carry

    lax.fori_loop(0, nchz, zdrain_body, 0)


def _params():
    return pltpu.CompilerParams(
        needs_layout_passes=False, use_tc_tiling_on_sc=False
    )


def _build_lists():
    mesh = plsc.VectorSubcoreMesh(core_axis_name="c", subcore_axis_name="s")
    return pl.kernel(
        _lists_body,
        out_type=(
            jax.ShapeDtypeStruct((NW, NL, LISTN), jnp.int32),
            jax.ShapeDtypeStruct((NW, 8), jnp.int32),
        ),
        mesh=mesh,
        scratch_types=[
            pltpu.VMEM((((OWN + 15) // 16) * 16,), jnp.int32),
            pltpu.VMEM((2 * ICH,), jnp.int32),
            pltpu.VMEM((2 * ICH,), jnp.int32),
            pltpu.VMEM((LISTN,), jnp.int32),
            pltpu.VMEM((LISTN,), jnp.int32),
            pltpu.VMEM((LISTN,), jnp.int32),
            pltpu.VMEM((LISTN,), jnp.int32),
            pltpu.VMEM((LISTN,), jnp.int32),
            pltpu.VMEM((16,), jnp.int32),
            pltpu.SemaphoreType.DMA,
        ],
        compiler_params=_params(),
    )


def _build_move():
    mesh = plsc.VectorSubcoreMesh(core_axis_name="c", subcore_axis_name="s")
    return pl.kernel(
        _move_body,
        out_type=jax.ShapeDtypeStruct((L, R, C), jnp.float32),
        mesh=mesh,
        scratch_types=[
            pltpu.VMEM((LISTN,), jnp.int32),
            pltpu.VMEM((LISTN,), jnp.int32),
            pltpu.VMEM((LISTN,), jnp.int32),
            pltpu.VMEM((LISTN,), jnp.int32),
            pltpu.VMEM((LISTN,), jnp.int32),
            pltpu.VMEM((16,), jnp.int32),
            pltpu.VMEM((ROWBUF, C), jnp.float32),
            pltpu.VMEM((RCH, C), jnp.float32),
            pltpu.SemaphoreType.DMA,
            pltpu.SemaphoreType.DMA,
            pltpu.SemaphoreType.DMA,
        ],
        compiler_params=_params(),
    )


def kernel(mat0, mat1, idx_from0, idx_to0, idx_from1, idx_to1):
    z = jnp.zeros((RCH, C), jnp.float32)
    lst, cnt = _build_lists()(
        idx_from0.astype(jnp.int32),
        idx_to0.astype(jnp.int32),
        idx_from1.astype(jnp.int32),
        idx_to1.astype(jnp.int32),
    )
    return _build_move()(mat0, mat1, lst, cnt, z)


# speedup vs baseline: 25.0444x; 1.0036x over previous
import jax
import jax.numpy as jnp
from jax import lax
from jax.experimental import pallas as pl
from jax.experimental.pallas import tpu as pltpu
from jax.experimental.pallas import tpu_sc as plsc

L, M, C = 2, 200000, 64
NI = 100000
R = 2 * NI
NW = 32
OWN = R // NW
ICH = 2000
NICH = NI // ICH
U = 5
GU = ICH // 16 // U
LISTN = 6528
RCH = 128
GRP = 5
ROWBUF = 2 * GRP * RCH
NL = 5


def _wid():
    return lax.axis_index("s") * 2 + lax.axis_index("c")


def _lists_body(if0, it0, if1, it1, lst, cnto, win, tbuf, fbuf,
                s0, d0, s1, d1, dz, cbuf, semt):
    wid = _wid()
    base = wid * OWN
    lanes = lax.iota(jnp.int32, 16)
    neg1 = jnp.zeros((16,), jnp.int32) - 1

    def init_body(k, carry):
        win[pl.ds(k * 16, 16)] = neg1
        return carry

    lax.fori_loop(0, (OWN + 15) // 16, init_body, 0)

    for s, it_hbm, if_hbm in ((0, it0, if0), (1, it1, if1)):

        def stream(ch, off):
            return (
                pltpu.make_async_copy(
                    it_hbm.at[pl.ds(ch * ICH, ICH)], tbuf.at[pl.ds(off, ICH)], semt
                ),
                pltpu.make_async_copy(
                    if_hbm.at[pl.ds(ch * ICH, ICH)], fbuf.at[pl.ds(off, ICH)], semt
                ),
            )

        for cp in stream(0, 0):
            cp.start()

        def chunk_body(ch, carry):
            off = (ch & 1) * ICH
            for cp in stream(ch, off):
                cp.wait()

            @pl.when(ch + 1 < NICH)
            def _():
                for cp in stream(ch + 1, ((ch + 1) & 1) * ICH):
                    cp.start()

            def vreg_body(v, c2):
                for u in range(U):
                    o = off + (v * U + u) * 16
                    t = tbuf[pl.ds(o, 16)]
                    f = fbuf[pl.ds(o, 16)]
                    rel = t - base
                    inb = (rel >= 0) & (rel < OWN)
                    g = f + s * M
                    _, lastm = plsc.scan_count(rel, mask=inb)
                    plsc.store_scatter(win, [rel], g, mask=lastm & inb)
                return c2

            lax.fori_loop(0, GU, vreg_body, 0)
            return carry

        lax.fori_loop(0, NICH, chunk_body, 0)

    def comp_body(v, carry):
        c0, c1, cz = carry
        g = win[pl.ds(v * 16, 16)]
        pos = v * 16 + lanes
        valid = pos < OWN
        dest = base + pos
        isz = g < 0
        is1 = g >= M
        m0 = valid & (~isz) & (~is1)
        m1 = valid & is1
        mz = valid & isz
        plsc.store_compressed(s0.at[pl.ds(c0, 16)], g, mask=m0)
        plsc.store_compressed(d0.at[pl.ds(c0, 16)], dest, mask=m0)
        plsc.store_compressed(s1.at[pl.ds(c1, 16)], g - M, mask=m1)
        plsc.store_compressed(d1.at[pl.ds(c1, 16)], dest, mask=m1)
        plsc.store_compressed(dz.at[pl.ds(cz, 16)], dest, mask=mz)
        c0 = c0 + jnp.max(plsc.all_reduce_population_count(m0))
        c1 = c1 + jnp.max(plsc.all_reduce_population_count(m1))
        cz = cz + jnp.max(plsc.all_reduce_population_count(mz))
        return c0, c1, cz

    nv = (OWN + 15) // 16
    c0, c1, cz = lax.fori_loop(
        0, nv, comp_body, (jnp.int32(0), jnp.int32(0), jnp.int32(0))
    )

    for svec, dvec, cnt in ((s0, d0, c0), (s1, d1, c1), (None, dz, cz)):
        dpad = jnp.zeros((16,), jnp.int32) + dvec[pl.ds(0, 16)][0]
        spad = (
            None
            if svec is None
            else jnp.zeros((16,), jnp.int32) + svec[pl.ds(0, 16)][0]
        )
        for k in range(RCH // 16):
            dvec[pl.ds(cnt + k * 16, 16)] = dpad
            if svec is not None:
                svec[pl.ds(cnt + k * 16, 16)] = spad

    for i, buf in enumerate((s0, d0, s1, d1, dz)):
        pltpu.sync_copy(buf, lst.at[wid].at[i])
    cv = jnp.where(lanes == 0, c0, jnp.where(lanes == 1, c1, jnp.where(lanes == 2, cz, 0)))
    cbuf[...] = cv
    pltpu.sync_copy(cbuf.at[pl.ds(0, 8)], cnto.at[wid])


def _move_body(a0, a1, lst, cnt, zhbm, out, s0, d0, s1, d1, dz, cbuf,
               rowbuf, zbuf, semg, sems, semz):
    wid = _wid()

    pltpu.sync_copy(zhbm, zbuf)
    for i, buf in enumerate((s0, d0, s1, d1, dz)):
        pltpu.sync_copy(lst.at[wid].at[i], buf)
    pltpu.sync_copy(cnt.at[wid], cbuf.at[pl.ds(0, 8)])
    cv = cbuf[pl.ds(0, 16)]
    c0 = cv[0]
    c1 = cv[1]
    cz = cv[2]

    nchz = (cz + RCH - 1) >> 7

    def zcopy(l, ci):
        return pltpu.make_async_copy(
            zbuf, out.at[l].at[dz.at[pl.ds(ci * RCH, RCH)]], semz
        )

    def zfire_body(ci, carry):
        for l in range(L):
            zcopy(l, ci).start()
        return carry

    lax.fori_loop(0, nchz, zfire_body, 0)

    for l in range(L):
        for src_hbm, svec, dvec, cn in ((a0, s0, d0, c0), (a1, s1, d1, c1)):
            nch = (cn + RCH - 1) >> 7
            ngrp = (nch + GRP - 1) // GRP

            def gcopy(ci, k, p):
                return pltpu.make_async_copy(
                    src_hbm.at[l].at[svec.at[pl.ds(ci * RCH, RCH)]],
                    rowbuf.at[pl.ds((p * GRP + k) * RCH, RCH)],
                    semg,
                )

            def scopy(ci, k, p):
                return pltpu.make_async_copy(
                    rowbuf.at[pl.ds((p * GRP + k) * RCH, RCH)],
                    out.at[l].at[dvec.at[pl.ds(ci * RCH, RCH)]],
                    sems,
                )

            def grp_body(gi, carry):
                @pl.when(gi >= 2)
                def _():
                    for k in range(GRP):
                        ci = (gi - 2) * GRP + k

                        @pl.when(ci < nch)
                        def _():
                            scopy(ci, k, gi & 1).wait()

                @pl.when(gi < ngrp)
                def _():
                    for k in range(GRP):
                        ci = gi * GRP + k

                        @pl.when(ci < nch)
                        def _():
                            gcopy(ci, k, gi & 1).start()

                    for k in range(GRP):
                        ci = gi * GRP + k

                        @pl.when(ci < nch)
                        def _():
                            gcopy(ci, k, gi & 1).wait()

                    for k in range(GRP):
                        ci = gi * GRP + k

                        @pl.when(ci < nch)
                        def _():
                            scopy(ci, k, gi & 1).start()

                return carry

            lax.fori_loop(0, ngrp + 2, grp_body, 0)

    def zdrain_body(ci, carry):
        for l in range(L):
            zcopy(l, ci).wait()
        return carry

    lax.fori_loop(0, nchz, zdrain_body, 0)


def _params():
    return pltpu.CompilerParams(
        needs_layout_passes=False, use_tc_tiling_on_sc=False
    )


def _build_lists():
    mesh = plsc.VectorSubcoreMesh(core_axis_name="c", subcore_axis_name="s")
    return pl.kernel(
        _lists_body,
        out_type=(
            jax.ShapeDtypeStruct((NW, NL, LISTN), jnp.int32),
            jax.ShapeDtypeStruct((NW, 8), jnp.int32),
        ),
        mesh=mesh,
        scratch_types=[
            pltpu.VMEM((((OWN + 15) // 16) * 16,), jnp.int32),
            pltpu.VMEM((2 * ICH,), jnp.int32),
            pltpu.VMEM((2 * ICH,), jnp.int32),
            pltpu.VMEM((LISTN,), jnp.int32),
            pltpu.VMEM((LISTN,), jnp.int32),
            pltpu.VMEM((LISTN,), jnp.int32),
            pltpu.VMEM((LISTN,), jnp.int32),
            pltpu.VMEM((LISTN,), jnp.int32),
            pltpu.VMEM((16,), jnp.int32),
            pltpu.SemaphoreType.DMA,
        ],
        compiler_params=_params(),
    )


def _build_move():
    mesh = plsc.VectorSubcoreMesh(core_axis_name="c", subcore_axis_name="s")
    return pl.kernel(
        _move_body,
        out_type=jax.ShapeDtypeStruct((L, R, C), jnp.float32),
        mesh=mesh,
        scratch_types=[
            pltpu.VMEM((LISTN,), jnp.int32),
            pltpu.VMEM((LISTN,), jnp.int32),
            pltpu.VMEM((LISTN,), jnp.int32),
            pltpu.VMEM((LISTN,), jnp.int32),
            pltpu.VMEM((LISTN,), jnp.int32),
            pltpu.VMEM((16,), jnp.int32),
            pltpu.VMEM((ROWBUF, C), jnp.float32),
            pltpu.VMEM((RCH, C), jnp.float32),
            pltpu.SemaphoreType.DMA,
            pltpu.SemaphoreType.DMA,
            pltpu.SemaphoreType.DMA,
        ],
        compiler_params=_params(),
    )


def kernel(mat0, mat1, idx_from0, idx_to0, idx_from1, idx_to1):
    z = jnp.zeros((RCH, C), jnp.float32)
    lst, cnt = _build_lists()(
        idx_from0.astype(jnp.int32),
        idx_to0.astype(jnp.int32),
        idx_from1.astype(jnp.int32),
        idx_to1.astype(jnp.int32),
    )
    return _build_move()(mat0, mat1, lst, cnt, z)
